# baseline (speedup 1.0000x reference)
import jax
import jax.numpy as jnp
import numpy as np
from jax.experimental import pallas as pl
from jax.experimental.pallas import tpu as pltpu
from jax.sharding import Mesh, PartitionSpec as P

N = 10000
IN_CH = 128
OUT_CH = 128
BM = 256


def _gcn_kernel(x_ref, w_ref, adj_ref, out_ref, support_ref):
    @pl.when(pl.program_id(0) == 0)
    def _():
        support_ref[...] = jnp.dot(
            x_ref[...], w_ref[...], preferred_element_type=jnp.float32
        )

    out_ref[...] = jnp.dot(
        adj_ref[...], support_ref[...], preferred_element_type=jnp.float32
    )


def _gcn_block(x, adj, W):
    rows = adj.shape[0]
    return pl.pallas_call(
        _gcn_kernel,
        grid=(pl.cdiv(rows, BM),),
        in_specs=[
            pl.BlockSpec((N, IN_CH), lambda i: (0, 0)),
            pl.BlockSpec((IN_CH, OUT_CH), lambda i: (0, 0)),
            pl.BlockSpec((BM, N), lambda i: (i, 0)),
        ],
        out_specs=pl.BlockSpec((BM, OUT_CH), lambda i: (i, 0)),
        out_shape=jax.ShapeDtypeStruct((rows, OUT_CH), jnp.float32),
        scratch_shapes=[pltpu.VMEM((N, OUT_CH), jnp.float32)],
    )(x, W, adj)


def kernel(x, adj, W):
    devs = jax.devices()
    ndev = 2 if len(devs) >= 2 else 1
    if ndev == 1:
        return _gcn_block(x, adj, W)
    mesh = Mesh(np.array(devs[:ndev]), ("i",))
    f = jax.shard_map(
        _gcn_block,
        mesh=mesh,
        in_specs=(P(None, None), P("i", None), P(None, None)),
        out_specs=P("i", None),
        check_vma=False,
    )
    return f(x, adj, W)


# speedup vs baseline: 5.6219x; 5.6219x over previous
import jax
import jax.numpy as jnp
from jax.experimental import pallas as pl
from jax.experimental.pallas import tpu as pltpu

N = 10000
IN_CH = 128
OUT_CH = 128
BM = 256


def _gcn_kernel(x_ref, w_ref, adj_ref, out_ref, support_ref):
    @pl.when(pl.program_id(0) == 0)
    def _():
        support_ref[...] = jnp.dot(
            x_ref[...], w_ref[...], preferred_element_type=jnp.float32
        )

    out_ref[...] = jnp.dot(
        adj_ref[...].astype(jnp.bfloat16),
        support_ref[...].astype(jnp.bfloat16),
        preferred_element_type=jnp.float32,
    )


@jax.jit
def kernel(x, adj, W):
    grid = (pl.cdiv(N, BM),)
    return pl.pallas_call(
        _gcn_kernel,
        grid=grid,
        in_specs=[
            pl.BlockSpec((N, IN_CH), lambda i: (0, 0)),
            pl.BlockSpec((IN_CH, OUT_CH), lambda i: (0, 0)),
            pl.BlockSpec((BM, N), lambda i: (i, 0)),
        ],
        out_specs=pl.BlockSpec((BM, OUT_CH), lambda i: (i, 0)),
        out_shape=jax.ShapeDtypeStruct((N, OUT_CH), jnp.float32),
        scratch_shapes=[pltpu.VMEM((N, OUT_CH), jnp.float32)],
    )(x, W, adj)


# speedup vs baseline: 5.6932x; 1.0127x over previous
import jax
import jax.numpy as jnp
from jax.experimental import pallas as pl
from jax.experimental.pallas import tpu as pltpu

N = 10000
IN_CH = 128
OUT_CH = 128
BM = 288


def _gcn_kernel(x_ref, w_ref, adj_ref, out_ref, support_ref):
    @pl.when(pl.program_id(0) == 0)
    def _():
        support_ref[...] = jnp.dot(
            x_ref[...], w_ref[...], preferred_element_type=jnp.float32
        )

    out_ref[...] = jnp.dot(
        adj_ref[...], support_ref[...], preferred_element_type=jnp.float32
    )


@jax.jit
def kernel(x, adj, W):
    grid = (pl.cdiv(N, BM),)
    return pl.pallas_call(
        _gcn_kernel,
        grid=grid,
        in_specs=[
            pl.BlockSpec((N, IN_CH), lambda i: (0, 0)),
            pl.BlockSpec((IN_CH, OUT_CH), lambda i: (0, 0)),
            pl.BlockSpec((BM, N), lambda i: (i, 0)),
        ],
        out_specs=pl.BlockSpec((BM, OUT_CH), lambda i: (i, 0)),
        out_shape=jax.ShapeDtypeStruct((N, OUT_CH), jnp.float32),
        scratch_shapes=[pltpu.VMEM((N, OUT_CH), jnp.float32)],
    )(x, W, adj)
